# baseline (speedup 1.0000x reference)
import functools
import jax
import jax.numpy as jnp
from jax import lax
from jax.experimental import pallas as pl
from jax.experimental.pallas import tpu as pltpu
from jax.experimental.pallas import tpu_sc as plsc

_HIDDEN = 128
_N = 4096
_RC = 4.6
_FACTOR = 0.5 * 27.211386024367243 * 0.5291772105638411
_WSUM = 1.875
_NTILES = 32
_PER_TILE = _N // _NTILES
_GROUPS = _PER_TILE // 16
_NGRP = _N // 16
_NGRP_PAD = 272


def _rsqrt_nr(d2):
    xi = lax.bitcast_convert_type(d2, jnp.int32)
    yi = jnp.int32(0x5F3759DF) - lax.shift_right_logical(xi, 1)
    y = lax.bitcast_convert_type(yi, jnp.float32)
    hd2 = 0.5 * d2
    y = y * (1.5 - hd2 * y * y)
    y = y * (1.5 - hd2 * y * y)
    return y


_GDN = lax.GatherDimensionNumbers(
    offset_dims=(), collapsed_slice_dims=(0,), start_index_map=(0,))


def _bcast(vec, kv):
    return lax.gather(vec, kv[:, None], _GDN, slice_sizes=(1,),
                      mode=lax.GatherScatterMode.PROMISE_IN_BOUNDS)


def _sc_coulomb_body(packed_h, lo_h, hi_h, out_h, pk, lo_v, hi_v, out_v):
    c = lax.axis_index("c")
    s = lax.axis_index("s")
    wid = s * 2 + c
    pltpu.sync_copy(packed_h, pk)
    pltpu.sync_copy(lo_h, lo_v)
    pltpu.sync_copy(hi_h, hi_v)

    inv_rc2 = 1.0 / (_RC * _RC)
    t2max = (1.0 - 1e-6) ** 2
    scale = _FACTOR / _WSUM
    base0 = pl.multiple_of(wid * _PER_TILE, _PER_TILE)
    bstart = pl.multiple_of(wid * _GROUPS, 8)
    lob = lo_v[pl.ds(bstart, 16)]
    hib = hi_v[pl.ds(bstart, 16)]

    for g in range(_GROUPS):
        base = pl.multiple_of(base0 + g * 16, 16)
        pxi = pk[0, pl.ds(base, 16)]
        pyi = pk[1, pl.ds(base, 16)]
        pzi = pk[2, pl.ds(base, 16)]
        q0i = pk[3, pl.ds(base, 16)] * scale
        q1i = pk[4, pl.ds(base, 16)] * scale
        q2i = pk[5, pl.ds(base, 16)] * scale
        q3i = pk[6, pl.ds(base, 16)] * scale
        bati = pk[7, pl.ds(base, 16)]
        jb_lo = lob[g]
        jb_hi = hib[g]

        def jb_body(jb, acc):
            js = pl.multiple_of(jb * 16, 16)
            pxj = pk[0, pl.ds(js, 16)]
            pyj = pk[1, pl.ds(js, 16)]
            pzj = pk[2, pl.ds(js, 16)]
            q0j = pk[3, pl.ds(js, 16)]
            q1j = pk[4, pl.ds(js, 16)]
            q2j = pk[5, pl.ds(js, 16)]
            q3j = pk[6, pl.ds(js, 16)]
            batj = pk[7, pl.ds(js, 16)]

            def pair(k, acc2):
                kv = jnp.full((16,), k, jnp.int32)
                dx = pxi - _bcast(pxj, kv)
                dy = pyi - _bcast(pyj, kv)
                dz = pzi - _bcast(pzj, kv)
                d2r = dx * dx + dy * dy + dz * dz
                d2 = jnp.where(d2r < 1e-12, 1e12, d2r)
                y = _rsqrt_nr(d2)
                t2 = jnp.minimum(d2 * inv_rc2, t2max)
                fc = 1.0 - jnp.exp(t2 / (t2 - 1.0))
                qq = (q0i * _bcast(q0j, kv) + q1i * _bcast(q1j, kv)
                      + q2i * _bcast(q2j, kv) + q3i * _bcast(q3j, kv))
                e = jnp.where(bati == _bcast(batj, kv), fc * qq * y, 0.0)
                return acc2 + e

            def k_body(k4, acc2):
                k = k4 * 4
                acc2 = pair(k, acc2)
                acc2 = pair(k + 1, acc2)
                acc2 = pair(k + 2, acc2)
                acc2 = pair(k + 3, acc2)
                return acc2

            return lax.fori_loop(0, 4, k_body, acc)

        acc = lax.fori_loop(jb_lo, jb_hi + 1, jb_body,
                            jnp.zeros((16,), jnp.float32))
        out_v[pl.ds(g * 16, 16)] = acc

    pltpu.sync_copy(out_v, out_h.at[pl.ds(base0, _PER_TILE)])


def _prep_mlp_body(xh_ref, xq_ref, pos_ref, bcol_ref, bt_ref, brow_ref,
                   W1_ref, b1_ref, W2_ref, b2_ref,
                   mlp_ref, packed_ref, lo_ref, hi_ref):
    hmid = jnp.dot(xh_ref[...], W1_ref[...],
                   preferred_element_type=jnp.float32) + b1_ref[...]
    hmid = hmid * jax.nn.sigmoid(hmid)
    mlp_ref[...] = jnp.dot(hmid, W2_ref[...],
                           preferred_element_type=jnp.float32) + b2_ref[...]

    q0 = xq_ref[:, 0:1]
    q1 = xq_ref[:, 1:2] * 0.7071067811865476
    q2 = xq_ref[:, 2:3] * 0.5
    q3 = xq_ref[:, 3:4] * 0.35355339059327373
    batf = bcol_ref[...].astype(jnp.float32)

    packed_ref[...] = lax.transpose(
        jnp.concatenate([pos_ref[...], q0, q1, q2, q3, batf], axis=1),
        (1, 0))

    b_first = bt_ref[:, 0:1]
    b_last = bt_ref[:, 15:16]
    br = brow_ref[...]
    lo_cnt = jnp.sum((br < b_first).astype(jnp.float32),
                     axis=1, keepdims=True)
    hi_cnt = jnp.sum((br <= b_last).astype(jnp.float32),
                     axis=1, keepdims=True)
    lo_blk = lax.shift_right_logical(lo_cnt.astype(jnp.int32), 4)
    hi_blk = lax.shift_right_logical(hi_cnt.astype(jnp.int32) - 1, 4)
    zpad = jnp.zeros((_NGRP_PAD - _NGRP, 1), jnp.int32)
    lo_ref[...] = jnp.concatenate([lo_blk, zpad], axis=0)
    hi_ref[...] = jnp.concatenate([hi_blk, zpad], axis=0)


def kernel(x, v, z, pos, batch, W1, b1, W2, b2):
    bcol = batch[:, None]
    bt = batch.reshape(_NGRP, 16)
    brow = batch[None, :]

    mlp, packed, lo2d, hi2d = pl.pallas_call(
        _prep_mlp_body,
        grid=(1,),
        in_specs=[
            pl.BlockSpec((_N, _HIDDEN), lambda i: (0, 0)),
            pl.BlockSpec((_N, 4), lambda i: (0, 0)),
            pl.BlockSpec((_N, 3), lambda i: (0, 0)),
            pl.BlockSpec((_N, 1), lambda i: (0, 0)),
            pl.BlockSpec((_NGRP, 16), lambda i: (0, 0)),
            pl.BlockSpec((1, _N), lambda i: (0, 0)),
            pl.BlockSpec((_HIDDEN, _HIDDEN // 2), lambda i: (0, 0)),
            pl.BlockSpec((1, _HIDDEN // 2), lambda i: (0, 0)),
            pl.BlockSpec((_HIDDEN // 2, 1), lambda i: (0, 0)),
            pl.BlockSpec((1, 1), lambda i: (0, 0)),
        ],
        out_specs=[
            pl.BlockSpec((_N, 1), lambda i: (0, 0)),
            pl.BlockSpec((8, _N), lambda i: (0, 0)),
            pl.BlockSpec((_NGRP_PAD, 1), lambda i: (0, 0)),
            pl.BlockSpec((_NGRP_PAD, 1), lambda i: (0, 0)),
        ],
        out_shape=[
            jax.ShapeDtypeStruct((_N, 1), jnp.float32),
            jax.ShapeDtypeStruct((8, _N), jnp.float32),
            jax.ShapeDtypeStruct((_NGRP_PAD, 1), jnp.int32),
            jax.ShapeDtypeStruct((_NGRP_PAD, 1), jnp.int32),
        ],
    )(x, x[:, _HIDDEN:], pos, bcol, bt, brow,
      W1, b1[None, :], W2, b2[None, :])

    mesh = plsc.VectorSubcoreMesh(core_axis_name="c", subcore_axis_name="s")
    sc_call = functools.partial(
        pl.kernel,
        mesh=mesh,
        out_type=jax.ShapeDtypeStruct((_N,), jnp.float32),
        scratch_types=[
            pltpu.VMEM((8, _N), jnp.float32),
            pltpu.VMEM((_NGRP_PAD,), jnp.int32),
            pltpu.VMEM((_NGRP_PAD,), jnp.int32),
            pltpu.VMEM((_PER_TILE,), jnp.float32),
        ],
    )(_sc_coulomb_body)
    e_i = sc_call(packed, lo2d.reshape(_NGRP_PAD), hi2d.reshape(_NGRP_PAD))

    return mlp + e_i[:, None]


# speedup vs baseline: 1.3862x; 1.3862x over previous
import functools
import jax
import jax.numpy as jnp
from jax import lax
from jax.experimental import pallas as pl
from jax.experimental.pallas import tpu as pltpu
from jax.experimental.pallas import tpu_sc as plsc

_HIDDEN = 128
_N = 4096
_RC = 4.6
_FACTOR = 0.5 * 27.211386024367243 * 0.5291772105638411
_WSUM = 1.875
_NTILES = 32
_PER_TILE = _N // _NTILES
_GROUPS = _PER_TILE // 16
_NGRP = _N // 16
_NGRP_PAD = 272


def _rsqrt_nr(d2):
    xi = lax.bitcast_convert_type(d2, jnp.int32)
    yi = jnp.int32(0x5F3759DF) - lax.shift_right_logical(xi, 1)
    y = lax.bitcast_convert_type(yi, jnp.float32)
    hd2 = 0.5 * d2
    y = y * (1.5 - hd2 * y * y)
    y = y * (1.5 - hd2 * y * y)
    return y


_GDN = lax.GatherDimensionNumbers(
    offset_dims=(), collapsed_slice_dims=(0,), start_index_map=(0,))


def _bcast(vec, kv):
    return lax.gather(vec, kv[:, None], _GDN, slice_sizes=(1,),
                      mode=lax.GatherScatterMode.PROMISE_IN_BOUNDS)


def _sc_coulomb_body(packed_h, lo_h, hi_h, out_h, pk, lo_v, hi_v, out_v):
    c = lax.axis_index("c")
    s = lax.axis_index("s")
    wid = s * 2 + c
    pltpu.sync_copy(packed_h, pk)
    pltpu.sync_copy(lo_h, lo_v)
    pltpu.sync_copy(hi_h, hi_v)

    inv_rc2 = 1.0 / (_RC * _RC)
    t2max = (1.0 - 1e-6) ** 2
    scale = _FACTOR / _WSUM
    base0 = pl.multiple_of(wid * _PER_TILE, _PER_TILE)
    bstart = pl.multiple_of(wid * _GROUPS, 8)
    lob = lo_v[pl.ds(bstart, 16)]
    hib = hi_v[pl.ds(bstart, 16)]

    for g in range(_GROUPS):
        base = pl.multiple_of(base0 + g * 16, 16)
        pxi = pk[0, pl.ds(base, 16)]
        pyi = pk[1, pl.ds(base, 16)]
        pzi = pk[2, pl.ds(base, 16)]
        q0i = pk[3, pl.ds(base, 16)] * scale
        q1i = pk[4, pl.ds(base, 16)] * scale
        q2i = pk[5, pl.ds(base, 16)] * scale
        q3i = pk[6, pl.ds(base, 16)] * scale
        bati = pk[7, pl.ds(base, 16)]
        jb_lo = lob[g]
        jb_hi = hib[g]

        def jb_body(jb, acc):
            js = pl.multiple_of(jb * 16, 16)
            pxj = pk[0, pl.ds(js, 16)]
            pyj = pk[1, pl.ds(js, 16)]
            pzj = pk[2, pl.ds(js, 16)]
            q0j = pk[3, pl.ds(js, 16)]
            q1j = pk[4, pl.ds(js, 16)]
            q2j = pk[5, pl.ds(js, 16)]
            q3j = pk[6, pl.ds(js, 16)]
            batj = pk[7, pl.ds(js, 16)]

            def pair(k, acc2):
                kv = jnp.full((16,), k, jnp.int32)
                dx = pxi - _bcast(pxj, kv)
                dy = pyi - _bcast(pyj, kv)
                dz = pzi - _bcast(pzj, kv)
                d2r = dx * dx + dy * dy + dz * dz
                d2 = jnp.where(d2r < 1e-12, 1e12, d2r)
                y = _rsqrt_nr(d2)
                t2 = jnp.minimum(d2 * inv_rc2, t2max)
                fc = 1.0 - jnp.exp(t2 / (t2 - 1.0))
                qq = (q0i * _bcast(q0j, kv) + q1i * _bcast(q1j, kv)
                      + q2i * _bcast(q2j, kv) + q3i * _bcast(q3j, kv))
                e = jnp.where(bati == _bcast(batj, kv), fc * qq * y, 0.0)
                return acc2 + e

            def k_body(k4, acc2):
                k = k4 * 4
                acc2 = pair(k, acc2)
                acc2 = pair(k + 1, acc2)
                acc2 = pair(k + 2, acc2)
                acc2 = pair(k + 3, acc2)
                return acc2

            return lax.fori_loop(0, 4, k_body, acc)

        acc = lax.fori_loop(jb_lo, jb_hi + 1, jb_body,
                            jnp.zeros((16,), jnp.float32))
        out_v[pl.ds(g * 16, 16)] = acc

    pltpu.sync_copy(out_v, out_h.at[pl.ds(base0, _PER_TILE)])


def _mlp_body(x_ref, W1_ref, b1_ref, W2_ref, b2_ref, out_ref):
    h = x_ref[:, :_HIDDEN]
    hmid = jnp.dot(h, W1_ref[...],
                   preferred_element_type=jnp.float32) + b1_ref[...]
    hmid = hmid * jax.nn.sigmoid(hmid)
    out_ref[...] = jnp.dot(hmid, W2_ref[...],
                           preferred_element_type=jnp.float32) + b2_ref[...]


def kernel(x, v, z, pos, batch, W1, b1, W2, b2):
    q = x[:, _HIDDEN:]
    sqw = jnp.array([1.0, 0.7071067811865476, 0.5, 0.35355339059327373],
                    dtype=jnp.float32)
    packed = jnp.concatenate(
        [pos, q * sqw, batch.astype(jnp.float32)[:, None]], axis=1).T

    b_first = batch[::16]
    b_last = batch[15::16]
    lo_atom = jnp.sum((batch[None, :] < b_first[:, None]).astype(jnp.int32),
                      axis=1)
    hi_atom = jnp.sum((batch[None, :] <= b_last[:, None]).astype(jnp.int32),
                      axis=1) - 1
    lo = jnp.pad(lo_atom // 16, (0, _NGRP_PAD - _NGRP)).astype(jnp.int32)
    hi = jnp.pad(hi_atom // 16, (0, _NGRP_PAD - _NGRP)).astype(jnp.int32)

    mesh = plsc.VectorSubcoreMesh(core_axis_name="c", subcore_axis_name="s")
    sc_call = functools.partial(
        pl.kernel,
        mesh=mesh,
        out_type=jax.ShapeDtypeStruct((_N,), jnp.float32),
        scratch_types=[
            pltpu.VMEM((8, _N), jnp.float32),
            pltpu.VMEM((_NGRP_PAD,), jnp.int32),
            pltpu.VMEM((_NGRP_PAD,), jnp.int32),
            pltpu.VMEM((_PER_TILE,), jnp.float32),
        ],
    )(_sc_coulomb_body)
    e_i = sc_call(packed, lo, hi)

    mlp = pl.pallas_call(
        _mlp_body,
        out_shape=jax.ShapeDtypeStruct((_N, 1), jnp.float32),
    )(x, W1, b1[None, :], W2, b2[None, :])

    return mlp + e_i[:, None]
